# Initial kernel scaffold; baseline (speedup 1.0000x reference)
import jax
import jax.numpy as jnp
from jax.experimental import pallas as pl


def kernel(x, edge_index, batch, W0, b0, W1, b1, W2, b2, W3, b3, cw1, cb1, cw2, cb2, dw1, db1, dw2, db2):
    raise NotImplementedError("write your pallas kernel here")


# speedup vs baseline: 1.1846x; 1.1846x over previous
import functools

import jax
import jax.numpy as jnp
from jax import lax
from jax.experimental import pallas as pl
from jax.experimental.pallas import tpu as pltpu
from jax.experimental.pallas import tpu_sc as plsc

N = 10000
E = 320000
NB = 64
HD = 128
KTOP = 30
TOT = 513
PADW = 640
NPAD = 10240
SENT = 10000

NC = 2
NS = 16
NW = NC * NS
EW = E // NW
CH = 128
NCHUNK = EW // CH
REM = EW - NCHUNK * CH
RPT = NPAD // NS

GN = NB * 32
GPW = GN // NW

RB = 1000
NGRID = N // RB


def _sc_mesh():
  return plsc.VectorSubcoreMesh(core_axis_name="c", subcore_axis_name="s",
                                num_cores=NC, num_subcores=NS)


def _make_sc_agg(width):
  @functools.partial(
      pl.kernel,
      out_type=jax.ShapeDtypeStruct((NC, NPAD, width), jnp.float32),
      mesh=_sc_mesh(),
      scratch_types=[
          pltpu.VMEM((CH,), jnp.int32),
          pltpu.VMEM((CH,), jnp.int32),
          pltpu.VMEM((CH, width), jnp.float32),
          pltpu.VMEM((REM,), jnp.int32),
          pltpu.VMEM((REM,), jnp.int32),
          pltpu.VMEM((REM, width), jnp.float32),
          pltpu.VMEM_SHARED((NPAD, width), jnp.float32),
          pltpu.SemaphoreType.DMA,
      ],
  )
  def agg(table, src, dst, zeros, out, sidx, didx, rows, sidx_r, didx_r,
          rows_r, acc, sem):
    cid = lax.axis_index("c")
    sid = lax.axis_index("s")
    wid = sid * NC + cid

    @pl.when(cid == 0)
    def _():
      pltpu.sync_copy(table.at[pl.ds(sid * RPT, RPT)],
                      acc.at[pl.ds(sid * RPT, RPT)])

    @pl.when(cid != 0)
    def _():
      pltpu.sync_copy(zeros.at[pl.ds(sid * RPT, RPT)],
                      acc.at[pl.ds(sid * RPT, RPT)])

    plsc.subcore_barrier()

    ebase = wid * EW

    def body(i, carry):
      base = ebase + i * CH
      pltpu.sync_copy(src.at[pl.ds(base, CH)], sidx)
      pltpu.sync_copy(dst.at[pl.ds(base, CH)], didx)
      pltpu.async_copy(table.at[sidx], rows, sem).wait()
      pltpu.sync_copy(rows, acc.at[didx], add=True)
      return carry

    lax.fori_loop(0, NCHUNK, body, 0)

    if REM:
      base = ebase + NCHUNK * CH
      pltpu.sync_copy(src.at[pl.ds(base, REM)], sidx_r)
      pltpu.sync_copy(dst.at[pl.ds(base, REM)], didx_r)
      pltpu.async_copy(table.at[sidx_r], rows_r, sem).wait()
      pltpu.sync_copy(rows_r, acc.at[didx_r], add=True)

    plsc.subcore_barrier()
    pltpu.sync_copy(acc.at[pl.ds(sid * RPT, RPT)],
                    out.at[cid, pl.ds(sid * RPT, RPT)])

  return agg


@functools.lru_cache(maxsize=None)
def _get_sc_agg(width):
  return _make_sc_agg(width)


@functools.lru_cache(maxsize=None)
def _get_sc_gather():
  @functools.partial(
      pl.kernel,
      out_type=jax.ShapeDtypeStruct((GN, PADW), jnp.float32),
      mesh=_sc_mesh(),
      scratch_types=[
          pltpu.VMEM((GPW,), jnp.int32),
          pltpu.VMEM((GPW, PADW), jnp.float32),
          pltpu.SemaphoreType.DMA,
      ],
  )
  def gather(zt, idx, out, idxv, rows, sem):
    cid = lax.axis_index("c")
    sid = lax.axis_index("s")
    wid = sid * NC + cid
    base = wid * GPW
    pltpu.sync_copy(idx.at[pl.ds(base, GPW)], idxv)
    pltpu.async_copy(zt.at[idxv], rows, sem).wait()
    pltpu.sync_copy(rows, out.at[pl.ds(base, GPW)])

  return gather


EG = E + N
EGPAD = 330240
GPW2 = EGPAD // NW


@functools.lru_cache(maxsize=None)
def _get_sc_gather_xw():
  nch = GPW2 // CH
  rem = GPW2 - nch * CH

  @functools.partial(
      pl.kernel,
      out_type=jax.ShapeDtypeStruct((EGPAD, HD), jnp.float32),
      mesh=_sc_mesh(),
      scratch_types=[
          pltpu.VMEM((CH,), jnp.int32),
          pltpu.VMEM((CH, HD), jnp.float32),
          pltpu.VMEM((rem,), jnp.int32),
          pltpu.VMEM((rem, HD), jnp.float32),
          pltpu.SemaphoreType.DMA,
      ],
  )
  def gather(xt, idx, out, idxv, rows, idxr, rowsr, sem):
    cid = lax.axis_index("c")
    sid = lax.axis_index("s")
    wid = sid * NC + cid
    base = wid * GPW2

    def body(i, carry):
      o = base + i * CH
      pltpu.sync_copy(idx.at[pl.ds(o, CH)], idxv)
      pltpu.async_copy(xt.at[idxv], rows, sem).wait()
      pltpu.sync_copy(rows, out.at[pl.ds(o, CH)])
      return carry

    lax.fori_loop(0, nch, body, 0)
    o = base + nch * CH
    pltpu.sync_copy(idx.at[pl.ds(o, rem)], idxr)
    pltpu.async_copy(xt.at[idxr], rowsr, sem).wait()
    pltpu.sync_copy(rowsr, out.at[pl.ds(o, rem)])

  return gather


def _prep_body(deg2_ref, x_ref, w_ref, dinv_ref, xw_ref):
  deg = deg2_ref[0] + deg2_ref[1]
  dinv = lax.rsqrt(deg[:, 0:1])
  dinv_ref[...] = jnp.broadcast_to(dinv, (RB, 8))
  xw_ref[...] = jnp.dot(x_ref[...], w_ref[...],
                        preferred_element_type=jnp.float32)


def _tc_prep(deg2, x, w0):
  return pl.pallas_call(
      _prep_body,
      grid=(NGRID,),
      in_specs=[
          pl.BlockSpec((NC, RB, HD), lambda i: (0, i, 0)),
          pl.BlockSpec((RB, HD), lambda i: (i, 0)),
          pl.BlockSpec((HD, HD), lambda i: (0, 0)),
      ],
      out_specs=[
          pl.BlockSpec((RB, 8), lambda i: (i, 0)),
          pl.BlockSpec((RB, HD), lambda i: (i, 0)),
      ],
      out_shape=[
          jax.ShapeDtypeStruct((N, 8), jnp.float32),
          jax.ShapeDtypeStruct((N, HD), jnp.float32),
      ],
  )(deg2, x, w0)


def _combine_body(agg_ref, b_ref, w_ref, h_ref, xw_ref):
  h = jnp.tanh(agg_ref[...] + b_ref[...])
  h_ref[...] = h
  xw_ref[...] = jnp.dot(h, w_ref[...], preferred_element_type=jnp.float32)


def _tc_combine(agg, b_prev, w_next):
  return pl.pallas_call(
      _combine_body,
      grid=(NGRID,),
      in_specs=[
          pl.BlockSpec((RB, HD), lambda i: (i, 0)),
          pl.BlockSpec((1, HD), lambda i: (0, 0)),
          pl.BlockSpec((HD, HD), lambda i: (0, 0)),
      ],
      out_specs=[
          pl.BlockSpec((RB, HD), lambda i: (i, 0)),
          pl.BlockSpec((RB, HD), lambda i: (i, 0)),
      ],
      out_shape=[
          jax.ShapeDtypeStruct((N, HD), jnp.float32),
          jax.ShapeDtypeStruct((N, HD), jnp.float32),
      ],
  )(agg, b_prev.reshape(1, HD), w_next)


def _tanh_add_body(agg_ref, b_ref, h_ref):
  h_ref[...] = jnp.tanh(agg_ref[...] + b_ref[...])


def _tc_tanh_add(agg, b_prev):
  return pl.pallas_call(
      _tanh_add_body,
      grid=(NGRID,),
      in_specs=[
          pl.BlockSpec((RB, HD), lambda i: (i, 0)),
          pl.BlockSpec((1, HD), lambda i: (0, 0)),
      ],
      out_specs=pl.BlockSpec((RB, HD), lambda i: (i, 0)),
      out_shape=jax.ShapeDtypeStruct((N, HD), jnp.float32),
  )(agg, b_prev.reshape(1, HD))


KROWS = NPAD // HD


def _topk_body(key_ref, batch_ref, out_ref):
  b = pl.program_id(0)
  kv = key_ref[...]
  mask = batch_ref[...] == b
  cnt = jnp.sum(mask.astype(jnp.int32))
  wk = jnp.where(mask, kv, -jnp.inf)
  gidx = (lax.broadcasted_iota(jnp.int32, (KROWS, HD), 0) * HD
          + lax.broadcasted_iota(jnp.int32, (KROWS, HD), 1))
  ids = []
  for j in range(KTOP):
    m = jnp.max(wk)
    am = jnp.min(jnp.where(wk == m, gidx, jnp.int32(2**30)))
    ids.append(jnp.where(j < cnt, am, SENT))
    wk = jnp.where(gidx == am, -jnp.inf, wk)
  lanes = lax.broadcasted_iota(jnp.int32, (1, 1, 32), 2)
  acc = jnp.full((1, 1, 32), SENT, jnp.int32)
  for j, v in enumerate(ids):
    acc = jnp.where(lanes == j, v, acc)
  out_ref[...] = acc


def _tc_topk(key2d, batch2d):
  return pl.pallas_call(
      _topk_body,
      grid=(NB,),
      in_specs=[
          pl.BlockSpec((KROWS, HD), lambda b: (0, 0)),
          pl.BlockSpec((KROWS, HD), lambda b: (0, 0)),
      ],
      out_specs=pl.BlockSpec((1, 1, 32), lambda b: (b, 0, 0)),
      out_shape=jax.ShapeDtypeStruct((NB, 1, 32), jnp.int32),
  )(key2d, batch2d)


C1 = 16
C2 = 32
KS = 5
L1 = KTOP - KS + 1
LP = L1 // 2
L2 = LP - KS + 1
DH = 128


def _cnn_body(p_ref, w1_ref, cb1_ref, w2_ref, cb2_ref, d1_ref, db1_ref,
              d2_ref, db2_ref, out_ref):
  P = p_ref[0]
  o1 = cb1_ref[...]
  acc1 = jnp.zeros((L1, C1), jnp.float32)
  for k in range(KS):
    acc1 = acc1 + jnp.dot(P[k:k + L1, :], w1_ref[k],
                          preferred_element_type=jnp.float32)
  o1 = jax.nn.relu(acc1 + o1)
  rows = [jnp.maximum(o1[2 * t:2 * t + 1, :], o1[2 * t + 1:2 * t + 2, :])
          for t in range(LP)]
  p1 = jnp.concatenate(rows, axis=0)
  acc2 = jnp.zeros((L2, C2), jnp.float32)
  for k in range(KS):
    acc2 = acc2 + jnp.dot(p1[k:k + L2, :], w2_ref[k],
                          preferred_element_type=jnp.float32)
  o2 = jax.nn.relu(acc2 + cb2_ref[...])
  accd = jnp.zeros((1, DH), jnp.float32)
  for t in range(L2):
    accd = accd + jnp.dot(o2[t:t + 1, :], d1_ref[t * C2:(t + 1) * C2, :],
                          preferred_element_type=jnp.float32)
  h = jax.nn.relu(accd + db1_ref[...])
  out_ref[0] = jnp.dot(h, d2_ref[...],
                       preferred_element_type=jnp.float32) + db2_ref[...]


def _tc_cnn(pooled, w1m, cb1, w2m, cb2, dw1p, db1, dw2p, db2p):
  return pl.pallas_call(
      _cnn_body,
      grid=(NB,),
      in_specs=[
          pl.BlockSpec((1, 32, PADW), lambda b: (b, 0, 0)),
          pl.BlockSpec((KS, PADW, C1), lambda b: (0, 0, 0)),
          pl.BlockSpec((1, C1), lambda b: (0, 0)),
          pl.BlockSpec((KS, C1, C2), lambda b: (0, 0, 0)),
          pl.BlockSpec((1, C2), lambda b: (0, 0)),
          pl.BlockSpec((L2 * C2, DH), lambda b: (0, 0)),
          pl.BlockSpec((1, DH), lambda b: (0, 0)),
          pl.BlockSpec((DH, 16), lambda b: (0, 0)),
          pl.BlockSpec((1, 16), lambda b: (0, 0)),
      ],
      out_specs=pl.BlockSpec((1, 1, 16), lambda b: (b, 0, 0)),
      out_shape=jax.ShapeDtypeStruct((NB, 1, 16), jnp.float32),
  )(pooled, w1m, cb1, w2m, cb2, dw1p, db1, dw2p, db2p)


def _pad_nodes(a):
  return jnp.pad(a, ((0, NPAD - N), (0, 0)))


def kernel(x, edge_index, batch, W0, b0, W1, b1, W2, b2, W3, b3, cw1, cb1,
           cw2, cb2, dw1, db1, dw2, db2):
  src = edge_index[0]
  dst = edge_index[1]
  loop = jnp.arange(N, dtype=src.dtype)
  s_full = jnp.concatenate([src, loop])
  d_full = jnp.concatenate([dst, loop])
  s_pad = jnp.pad(s_full, (0, EGPAD - EG))

  zeros128 = jnp.zeros((NPAD, HD), jnp.float32)
  ones128 = jnp.pad(jnp.ones((N, HD), jnp.float32), ((0, NPAD - N), (0, 0)))

  sc_agg128 = _get_sc_agg(HD)
  sc_gxw = _get_sc_gather_xw()

  deg2 = sc_agg128(ones128, src, dst, zeros128)[:, :N]

  dinv8, xw = _tc_prep(deg2, x, W0)
  dinv = dinv8[:, 0]
  norm = dinv[s_full] * dinv[d_full]

  hs = []
  bs = [b0, b1, b2]
  ws = [W1, W2]
  for i in range(3):
    hg = sc_gxw(_pad_nodes(xw), s_pad)[:EG]
    msgs = hg * norm[:, None]
    agg = jax.ops.segment_sum(msgs, d_full, num_segments=N)
    if i < 2:
      h, xw = _tc_combine(agg, bs[i], ws[i])
    else:
      h = _tc_tanh_add(agg, bs[i])
    hs.append(h)
  h1, h2, h3 = hs

  xw3 = (h3 @ W3)[:, 0]
  key = jax.ops.segment_sum(xw3[s_full] * norm, d_full,
                            num_segments=N) + b3[0]

  z = jnp.concatenate(
      [x, h1, h2, h3, key[:, None],
       jnp.zeros((N, PADW - TOT), jnp.float32)], axis=1)
  zp = _pad_nodes(z)

  keyp = jnp.pad(key, (0, NPAD - N)).reshape(KROWS, HD)
  batchp = jnp.pad(batch, (0, NPAD - N),
                   constant_values=NB).reshape(KROWS, HD)
  idx = _tc_topk(keyp, batchp).reshape(GN)

  pooled = _get_sc_gather()(zp, idx).reshape(NB, 32, PADW)

  w1m = jnp.pad(jnp.transpose(cw1, (2, 1, 0)),
                ((0, 0), (0, PADW - TOT), (0, 0)))
  w2m = jnp.transpose(cw2, (2, 1, 0))
  dw1p = dw1.reshape(C2, L2, DH).swapaxes(0, 1).reshape(L2 * C2, DH)
  dw2p = jnp.pad(dw2, ((0, 0), (0, 6)))
  db2p = jnp.pad(db2, (0, 6))

  out = _tc_cnn(pooled, w1m, cb1.reshape(1, C1), w2m, cb2.reshape(1, C2),
                dw1p, db1.reshape(1, DH), dw2p, db2p.reshape(1, 16))
  return out[:, 0, :10]
